# baseline (speedup 1.0000x reference)
import functools

import jax
import jax.numpy as jnp
from jax.experimental import pallas as pl

_EPS = 1e-6
_D = 64
_NSYM = 32
_DEPTH = 8
_BLK = 512


def _crsn_body(x_ref, em_ref, ep_ref, wr_ref, wi_ref, lns_ref, lnb_ref,
               mb_ref, hw_ref, hb_ref, cb_ref, adj_ref, dw_ref, db_ref,
               logits_ref, feats_ref, sym_ref, stats_ref):
    i = pl.program_id(0)

    @pl.when(i == 0)
    def _():
        stats_ref[...] = jnp.zeros_like(stats_ref)

    f32 = jnp.float32
    bf16 = jnp.bfloat16
    contract1 = (((1,), (1,)), ((), ()))
    blk = x_ref.shape[0]
    iota_sym = jax.lax.broadcasted_iota(jnp.int32, (blk, _NSYM), 1)
    iota_f = iota_sym.astype(jnp.float32)

    def split3(m):
        m1 = m.astype(bf16).astype(f32)
        r1 = m - m1
        m2 = r1.astype(bf16).astype(f32)
        return m1, m2, r1 - m2

    def gather(oh, parts):
        out = jnp.dot(oh, parts[0], preferred_element_type=f32)
        for p in parts[1:]:
            out = out + jnp.dot(oh, p, preferred_element_type=f32)
        return out

    xb = x_ref[:, 0]
    ohx = (iota_sym == xb[:, None]).astype(f32)
    em = em_ref[...]
    ep = ep_ref[...]
    table = jnp.concatenate([em * jnp.cos(ep), em * jnp.sin(ep)], axis=1)
    zf = gather(ohx, split3(table))

    wr = wr_ref[...]
    wi = wi_ref[...]
    n_mat = jnp.concatenate(
        [jnp.concatenate([wr, -wi], axis=1),
         jnp.concatenate([wi, wr], axis=1)], axis=0)

    cb = cb_ref[...]
    cb_sq = jnp.sum(cb * cb, axis=1)[None, :]
    cb_parts = split3(cb)
    sadj = 0.1 * jax.nn.sigmoid(adj_ref[...])
    sadj1 = sadj.astype(bf16).astype(f32)
    adj_parts = (sadj1, sadj - sadj1)
    cbh = jnp.concatenate([-2.0 * cb, hw_ref[...]], axis=0)
    hb = hb_ref[0, 0]
    lns = lns_ref[...]
    lnb = lnb_ref[...]
    mb = mb_ref[...]
    onezero = jnp.concatenate(
        [jnp.ones((1, _D), f32), jnp.zeros((1, _D), f32)], axis=1)

    hp = jnp.zeros((blk, 1), f32)
    rem = jnp.ones((blk, 1), f32)
    za = jnp.zeros((blk, 2 * _D), f32)
    still_acc = jnp.zeros((blk, 1), f32)
    vq_acc = jnp.zeros((blk, 2 * _D), f32)

    def step(t, zf, oh_prev, hp, rem, za, still_acc, vq_acc):
        nrni = jax.lax.dot_general(zf, n_mat, contract1,
                                   preferred_element_type=f32)
        sq = nrni * nrni
        hyp2 = sq + jnp.concatenate([sq[:, _D:], sq[:, :_D]], axis=1)
        safe = hyp2 > 0.0
        inv = jnp.where(safe, jax.lax.rsqrt(hyp2), 0.0)
        mag = hyp2 * inv + _EPS

        s1 = jnp.sum(mag, axis=1, keepdims=True)
        mean = s1 * (1.0 / (2 * _D))
        dev = mag - mean
        s2 = jnp.sum(dev * dev, axis=1, keepdims=True)
        var = s2 * (1.0 / (2 * (_D - 1)))
        mn = (dev * jax.lax.rsqrt(var + _EPS)) * lns + lnb

        cs = jnp.where(safe, nrni * inv, onezero)
        zf = mn * cs

        mag2 = jnp.abs(mn) + _EPS
        sc = jnp.maximum(mag2 + mb, 0.0) / mag2
        zf = zf * sc

        scores_all = jax.lax.dot_general(zf, cbh, contract1,
                                         preferred_element_type=f32)
        p = jax.nn.sigmoid(scores_all[:, _NSYM:_NSYM + 1] + hb)

        dist = cb_sq + scores_all[:, :_NSYM]
        if oh_prev is None:
            dadj = dist
        else:
            dadj = dist - gather(oh_prev, adj_parts)
        minv = jnp.min(dadj, axis=1, keepdims=True)
        cand = jnp.where(dadj <= minv, iota_f, float(_NSYM))
        idx = jnp.min(cand, axis=1, keepdims=True)
        oh = (iota_f == idx).astype(f32)

        zq = gather(oh, cb_parts)
        dq = zq - zf
        vq_acc = vq_acc + dq * dq

        zf = 0.7 * zf + 0.3 * zq

        still = (hp < 0.99).astype(f32)
        last = t == _DEPTH - 1
        p_eff = jnp.where(last, rem, p * still)
        za = za + p_eff * zf
        hp = hp + p_eff
        rem = rem - p_eff
        still_acc = still_acc + still
        return zf, oh, idx, hp, rem, za, still_acc, vq_acc

    zf, oh, idx, hp, rem, za, still_acc, vq_acc = step(
        0, zf, None, hp, rem, za, still_acc, vq_acc)

    for t in range(1, _DEPTH):
        zf, oh, idx, hp, rem, za, still_acc, vq_acc = step(
            t, zf, oh, hp, rem, za, still_acc, vq_acc)

    logits = jax.lax.dot_general(za, dw_ref[...], contract1,
                                 preferred_element_type=f32) + db_ref[...]
    logits_ref[...] = logits
    feats_ref[...] = za
    sym_ref[...] = idx.astype(jnp.int32)
    ponder = jnp.sum(still_acc, axis=(0, 1), keepdims=True)
    vqs = jnp.sum(vq_acc, axis=(0, 1), keepdims=True)
    stats_ref[...] += jnp.concatenate([ponder, vqs], axis=1)


@functools.partial(jax.jit, static_argnames=("interpret",))
def _run(x, emb_mag, emb_phase, Wr, Wi, ln_scale, ln_shift, mod_bias,
         halt_W, halt_b, codebook, adj, dec_W, dec_b, interpret=False):
    batch = x.shape[0]
    vocab, d = emb_mag.shape
    nb = batch // _BLK

    x2 = x.astype(jnp.int32).reshape(batch, 1)
    em_p = jnp.zeros((_NSYM, d), jnp.float32).at[:vocab].set(emb_mag)
    ep_p = jnp.zeros((_NSYM, d), jnp.float32).at[:vocab].set(emb_phase)
    dw_p = jnp.zeros((_NSYM, 2 * d), jnp.float32).at[:dec_W.shape[0]].set(dec_W)
    db_p = jnp.zeros((1, _NSYM), jnp.float32).at[0, :dec_b.shape[0]].set(dec_b)
    lns2 = jnp.concatenate([ln_scale, ln_scale]).reshape(1, 2 * d)
    lnb2 = jnp.concatenate([ln_shift, ln_shift]).reshape(1, 2 * d)
    mb2 = jnp.concatenate([mod_bias, mod_bias]).reshape(1, 2 * d)
    hw8 = jnp.zeros((8, 2 * d), jnp.float32).at[:1].set(halt_W.reshape(1, 2 * d))

    full = lambda shape: pl.BlockSpec(shape, lambda i: (0, 0))
    out = pl.pallas_call(
        _crsn_body,
        grid=(nb,),
        in_specs=[
            pl.BlockSpec((_BLK, 1), lambda i: (i, 0)),
            full((_NSYM, d)), full((_NSYM, d)),
            full((d, d)), full((d, d)),
            full((1, 2 * d)), full((1, 2 * d)), full((1, 2 * d)),
            full((8, 2 * d)), full((1, 1)),
            full((_NSYM, 2 * d)), full((_NSYM, _NSYM)),
            full((_NSYM, 2 * d)), full((1, _NSYM)),
        ],
        out_specs=[
            pl.BlockSpec((_BLK, _NSYM), lambda i: (i, 0)),
            pl.BlockSpec((_BLK, 2 * d), lambda i: (i, 0)),
            pl.BlockSpec((_BLK, 1), lambda i: (i, 0)),
            pl.BlockSpec((1, 2), lambda i: (0, 0)),
        ],
        out_shape=[
            jax.ShapeDtypeStruct((batch, _NSYM), jnp.float32),
            jax.ShapeDtypeStruct((batch, 2 * d), jnp.float32),
            jax.ShapeDtypeStruct((batch, 1), jnp.int32),
            jax.ShapeDtypeStruct((1, 2), jnp.float32),
        ],
        interpret=interpret,
    )(x2, em_p, ep_p, Wr, Wi, lns2, lnb2, mb2, hw8,
      halt_b.reshape(1, 1).astype(jnp.float32), codebook, adj, dw_p, db_p)

    logits_p, feats, sym2, stats = out
    logits = logits_p[:, :dec_W.shape[0]]
    z_accum = jax.lax.complex(feats[:, :d], feats[:, d:])
    sym = sym2[:, 0]
    ponder = stats[0, 0] / batch
    vq_total = stats[0, 1] * (1.25 / (batch * 2 * d))
    return (logits, z_accum, sym, ponder, vq_total)


def kernel(x, emb_mag, emb_phase, Wr, Wi, ln_scale, ln_shift, mod_bias,
           halt_W, halt_b, codebook, adj, dec_W, dec_b):
    return _run(x, emb_mag, emb_phase, Wr, Wi, ln_scale, ln_shift, mod_bias,
                halt_W, halt_b, codebook, adj, dec_W, dec_b)


# speedup vs baseline: 1.0725x; 1.0725x over previous
import functools

import jax
import jax.numpy as jnp
from jax.experimental import pallas as pl

_EPS = 1e-6
_D = 64
_NSYM = 32
_DEPTH = 8
_BLK = 1024


def _crsn_body(x_ref, em_ref, ep_ref, wr_ref, wi_ref, lns_ref, lnb_ref,
               mb_ref, hw_ref, hb_ref, cb_ref, adj_ref, dw_ref, db_ref,
               logits_ref, feats_ref, sym_ref, stats_ref):
    i = pl.program_id(0)

    @pl.when(i == 0)
    def _():
        stats_ref[...] = jnp.zeros_like(stats_ref)

    f32 = jnp.float32
    bf16 = jnp.bfloat16
    contract1 = (((1,), (1,)), ((), ()))
    blk = x_ref.shape[0]
    iota_sym = jax.lax.broadcasted_iota(jnp.int32, (blk, _NSYM), 1)
    iota_f = iota_sym.astype(jnp.float32)

    def split3(m):
        m1 = m.astype(bf16).astype(f32)
        r1 = m - m1
        m2 = r1.astype(bf16).astype(f32)
        return m1, m2, r1 - m2

    def gather(oh, parts):
        out = jnp.dot(oh, parts[0], preferred_element_type=f32)
        for p in parts[1:]:
            out = out + jnp.dot(oh, p, preferred_element_type=f32)
        return out

    xb = x_ref[:, 0]
    ohx = (iota_sym == xb[:, None]).astype(f32)
    em = em_ref[...]
    ep = ep_ref[...]
    table = jnp.concatenate([em * jnp.cos(ep), em * jnp.sin(ep)], axis=1)
    zf = gather(ohx, split3(table))

    wr = wr_ref[...]
    wi = wi_ref[...]
    n_mat = jnp.concatenate(
        [jnp.concatenate([wr, -wi], axis=1),
         jnp.concatenate([wi, wr], axis=1)], axis=0)

    cb = cb_ref[...]
    cb_sq = jnp.sum(cb * cb, axis=1)[None, :]
    cb_parts = split3(cb)
    sadj = 0.1 * jax.nn.sigmoid(adj_ref[...])
    sadj1 = sadj.astype(bf16).astype(f32)
    adj_parts = (sadj1, sadj - sadj1)
    cbh = jnp.concatenate([-2.0 * cb, hw_ref[...]], axis=0)
    hb = hb_ref[0, 0]
    lns = lns_ref[...]
    lnb = lnb_ref[...]
    mb = mb_ref[...]
    onezero = jnp.concatenate(
        [jnp.ones((1, _D), f32), jnp.zeros((1, _D), f32)], axis=1)

    hp = jnp.zeros((blk, 1), f32)
    rem = jnp.ones((blk, 1), f32)
    za = jnp.zeros((blk, 2 * _D), f32)
    still_acc = jnp.zeros((blk, 1), f32)
    vq_acc = jnp.zeros((blk, 2 * _D), f32)

    def step(t, zf, oh_prev, hp, rem, za, still_acc, vq_acc):
        nrni = jax.lax.dot_general(zf, n_mat, contract1,
                                   preferred_element_type=f32)
        sq = nrni * nrni
        hyp2 = sq + jnp.concatenate([sq[:, _D:], sq[:, :_D]], axis=1)
        safe = hyp2 > 0.0
        inv = jnp.where(safe, jax.lax.rsqrt(hyp2), 0.0)
        mag = hyp2 * inv + _EPS

        s1 = jnp.sum(mag, axis=1, keepdims=True)
        mean = s1 * (1.0 / (2 * _D))
        dev = mag - mean
        s2 = jnp.sum(dev * dev, axis=1, keepdims=True)
        var = s2 * (1.0 / (2 * (_D - 1)))
        mn = (dev * jax.lax.rsqrt(var + _EPS)) * lns + lnb

        cs = jnp.where(safe, nrni * inv, onezero)
        zf = mn * cs

        mag2 = jnp.abs(mn) + _EPS
        sc = jnp.maximum(mag2 + mb, 0.0) / mag2
        zf = zf * sc

        scores_all = jax.lax.dot_general(zf, cbh, contract1,
                                         preferred_element_type=f32)
        p = jax.nn.sigmoid(scores_all[:, _NSYM:_NSYM + 1] + hb)

        dist = cb_sq + scores_all[:, :_NSYM]
        if oh_prev is None:
            dadj = dist
        else:
            dadj = dist - gather(oh_prev, adj_parts)
        minv = jnp.min(dadj, axis=1, keepdims=True)
        cand = jnp.where(dadj <= minv, iota_f, float(_NSYM))
        idx = jnp.min(cand, axis=1, keepdims=True)
        oh = (iota_f == idx).astype(f32)

        zq = gather(oh, cb_parts)
        dq = zq - zf
        vq_acc = vq_acc + dq * dq

        zf = 0.7 * zf + 0.3 * zq

        still = (hp < 0.99).astype(f32)
        last = t == _DEPTH - 1
        p_eff = jnp.where(last, rem, p * still)
        za = za + p_eff * zf
        hp = hp + p_eff
        rem = rem - p_eff
        still_acc = still_acc + still
        return zf, oh, idx, hp, rem, za, still_acc, vq_acc

    zf, oh, idx, hp, rem, za, still_acc, vq_acc = step(
        0, zf, None, hp, rem, za, still_acc, vq_acc)

    for t in range(1, _DEPTH):
        zf, oh, idx, hp, rem, za, still_acc, vq_acc = step(
            t, zf, oh, hp, rem, za, still_acc, vq_acc)

    logits = jax.lax.dot_general(za, dw_ref[...], contract1,
                                 preferred_element_type=f32) + db_ref[...]
    logits_ref[...] = logits
    feats_ref[...] = za
    sym_ref[...] = idx.astype(jnp.int32)
    ponder = jnp.sum(still_acc, axis=(0, 1), keepdims=True)
    vqs = jnp.sum(vq_acc, axis=(0, 1), keepdims=True)
    stats_ref[...] += jnp.concatenate([ponder, vqs], axis=1)


@functools.partial(jax.jit, static_argnames=("interpret",))
def _run(x, emb_mag, emb_phase, Wr, Wi, ln_scale, ln_shift, mod_bias,
         halt_W, halt_b, codebook, adj, dec_W, dec_b, interpret=False):
    batch = x.shape[0]
    vocab, d = emb_mag.shape
    nb = batch // _BLK

    x2 = x.astype(jnp.int32).reshape(batch, 1)
    em_p = jnp.zeros((_NSYM, d), jnp.float32).at[:vocab].set(emb_mag)
    ep_p = jnp.zeros((_NSYM, d), jnp.float32).at[:vocab].set(emb_phase)
    dw_p = jnp.zeros((_NSYM, 2 * d), jnp.float32).at[:dec_W.shape[0]].set(dec_W)
    db_p = jnp.zeros((1, _NSYM), jnp.float32).at[0, :dec_b.shape[0]].set(dec_b)
    lns2 = jnp.concatenate([ln_scale, ln_scale]).reshape(1, 2 * d)
    lnb2 = jnp.concatenate([ln_shift, ln_shift]).reshape(1, 2 * d)
    mb2 = jnp.concatenate([mod_bias, mod_bias]).reshape(1, 2 * d)
    hw8 = jnp.zeros((8, 2 * d), jnp.float32).at[:1].set(halt_W.reshape(1, 2 * d))

    full = lambda shape: pl.BlockSpec(shape, lambda i: (0, 0))
    out = pl.pallas_call(
        _crsn_body,
        grid=(nb,),
        in_specs=[
            pl.BlockSpec((_BLK, 1), lambda i: (i, 0)),
            full((_NSYM, d)), full((_NSYM, d)),
            full((d, d)), full((d, d)),
            full((1, 2 * d)), full((1, 2 * d)), full((1, 2 * d)),
            full((8, 2 * d)), full((1, 1)),
            full((_NSYM, 2 * d)), full((_NSYM, _NSYM)),
            full((_NSYM, 2 * d)), full((1, _NSYM)),
        ],
        out_specs=[
            pl.BlockSpec((_BLK, _NSYM), lambda i: (i, 0)),
            pl.BlockSpec((_BLK, 2 * d), lambda i: (i, 0)),
            pl.BlockSpec((_BLK, 1), lambda i: (i, 0)),
            pl.BlockSpec((1, 2), lambda i: (0, 0)),
        ],
        out_shape=[
            jax.ShapeDtypeStruct((batch, _NSYM), jnp.float32),
            jax.ShapeDtypeStruct((batch, 2 * d), jnp.float32),
            jax.ShapeDtypeStruct((batch, 1), jnp.int32),
            jax.ShapeDtypeStruct((1, 2), jnp.float32),
        ],
        interpret=interpret,
    )(x2, em_p, ep_p, Wr, Wi, lns2, lnb2, mb2, hw8,
      halt_b.reshape(1, 1).astype(jnp.float32), codebook, adj, dw_p, db_p)

    logits_p, feats, sym2, stats = out
    logits = logits_p[:, :dec_W.shape[0]]
    z_accum = jax.lax.complex(feats[:, :d], feats[:, d:])
    sym = sym2[:, 0]
    ponder = stats[0, 0] / batch
    vq_total = stats[0, 1] * (1.25 / (batch * 2 * d))
    return (logits, z_accum, sym, ponder, vq_total)


def kernel(x, emb_mag, emb_phase, Wr, Wi, ln_scale, ln_shift, mod_bias,
           halt_W, halt_b, codebook, adj, dec_W, dec_b):
    return _run(x, emb_mag, emb_phase, Wr, Wi, ln_scale, ln_shift, mod_bias,
                halt_W, halt_b, codebook, adj, dec_W, dec_b)
